# baseline (speedup 1.0000x reference)
import functools

import jax
import jax.numpy as jnp
from jax import lax
from jax.experimental import pallas as pl
from jax.experimental.pallas import tpu as pltpu
from jax.experimental.pallas import tpu_sc as plsc

_B = 4
_N = 5000
_NPAD = 5120
_BLKS = _NPAD // 128
_SEG = _NPAD // 8
_SEGCH = 40
_CAP = _SEG + 16
_MAXDET = 100
_MINCONF = 0.7
_NMS_T = 0.3
_BIG = jnp.int32(1 << 20)


def _prep_kernel(rois_ref, cls_ref, out_ref):
    y1 = rois_ref[:, 0]
    x1 = rois_ref[:, 1]
    y2 = rois_ref[:, 2]
    x2 = rois_ref[:, 3]
    dy = cls_ref[:, 0] * 0.1
    dx = cls_ref[:, 1] * 0.1
    dh = cls_ref[:, 2] * 0.2
    dw = cls_ref[:, 3] * 0.2
    cls_f = cls_ref[:, 4]
    raw_scores = cls_ref[:, 5]

    h = y2 - y1
    w = x2 - x1
    cy = y1 + 0.5 * h + dy * h
    cx = x1 + 0.5 * w + dx * w
    h = h * jnp.exp(dh)
    w = w * jnp.exp(dw)
    ry1 = jnp.clip(cy - 0.5 * h, 0.0, 1.0)
    rx1 = jnp.clip(cx - 0.5 * w, 0.0, 1.0)
    ry2 = jnp.clip((cy - 0.5 * h) + h, 0.0, 1.0)
    rx2 = jnp.clip((cx - 0.5 * w) + w, 0.0, 1.0)

    cls_i = cls_f.astype(jnp.int32)
    keep = (cls_i > 0) & (raw_scores >= _MINCONF)
    scores = jnp.where(keep, raw_scores, -1.0)

    off = cls_f * 4.0
    out_ref[:, 0] = ry1 + off
    out_ref[:, 1] = rx1 + off
    out_ref[:, 2] = ry2 + off
    out_ref[:, 3] = rx2 + off
    out_ref[:, 4] = cls_f
    out_ref[:, 5] = scores


def _nms_sc(cand_hbm, out_hbm, *refs):
    seg = refs[0:6]
    cch = refs[6:12]
    msg_v = refs[12]
    slots_v = refs[13]
    det = refs[14:20]
    slots_sh = refs[20]
    c = lax.axis_index("c")
    s = lax.axis_index("s")
    g = s // 8
    slot = s % 8
    batch = c * 2 + g
    row = g * 8 + slot
    iota = lax.iota(jnp.int32, 16)

    for k in range(6):
        pltpu.sync_copy(
            cand_hbm.at[batch, k, pl.ds(slot * _SEG, _SEG)], seg[k]
        )

    def compact_body(j, cnt):
        idxv = j * 16 + iota
        sc = plsc.load_gather(seg[5], [idxv])
        m = sc > 0.0
        incl = plsc.cumsum(jnp.where(m, 1, 0))
        pos = cnt + incl - 1
        for k in range(5):
            v = plsc.load_gather(seg[k], [idxv])
            plsc.store_scatter(cch[k], [pos], v, mask=m)
        plsc.store_scatter(cch[5], [pos], sc, mask=m)
        return cnt + jnp.max(incl)

    cnt = lax.fori_loop(0, _SEGCH, compact_body, jnp.int32(0))
    plsc.store_scatter(
        cch[5], [cnt + iota], jnp.full((16,), -1.0, jnp.float32)
    )
    nchunks = (cnt + 15) // 16

    @pl.when(slot == 0)
    def _():
        for k in range(6):
            for j in range(8):
                det[k][pl.ds(j * 16, 16)] = jnp.zeros((16,), jnp.float32)

    neg16 = jnp.full((16,), -1.0, jnp.float32)

    def nms_body(i, _):
        def am_body(j, carry):
            bv, bi = carry
            idxv = j * 16 + iota
            sc = plsc.load_gather(cch[5], [idxv])
            better = (sc > bv) | ((sc == bv) & (idxv < bi))
            return jnp.where(better, sc, bv), jnp.where(better, idxv, bi)

        bv, bi = lax.fori_loop(
            0, nchunks, am_body, (neg16, jnp.full((16,), _BIG))
        )
        m = jnp.max(bv)
        ii = jnp.min(jnp.where(bv == m, bi, _BIG))
        ii_safe = jnp.minimum(ii, jnp.int32(_SEG - 1))
        iis = jnp.full((16,), ii_safe, jnp.int32)

        msg = jnp.where(iota == 5, m, 0.0)
        for k in range(5):
            wv = plsc.load_gather(cch[k], [iis])
            msg = jnp.where(iota == k, wv, msg)
        msg_v[...] = msg
        pltpu.sync_copy(msg_v, slots_sh.at[pl.ds(row * 16, 16)])
        plsc.subcore_barrier()
        pltpu.sync_copy(slots_sh.at[pl.ds(g * 128, 128)], slots_v)

        win_row = slots_v[pl.ds(0, 16)]
        win_sc = win_row[5]
        win_slot = jnp.int32(0)
        for w in range(1, 8):
            roww = slots_v[pl.ds(w * 16, 16)]
            sw = roww[5]
            better = sw > win_sc
            win_slot = jnp.where(better, jnp.int32(w), win_slot)
            win_sc = jnp.where(better, sw, win_sc)
            win_row = jnp.where(better, roww, win_row)
        alive = win_sc > 0.0

        wy1 = win_row[0]
        wx1 = win_row[1]
        wy2 = win_row[2]
        wx2 = win_row[3]
        wcls = win_row[4]
        warea = (wy2 - wy1) * (wx2 - wx1)

        @pl.when(alive)
        def _():
            wy1v = jnp.full((16,), wy1)
            wx1v = jnp.full((16,), wx1)
            wy2v = jnp.full((16,), wy2)
            wx2v = jnp.full((16,), wx2)
            wareav = jnp.full((16,), warea)

            def sup_body(j, _):
                idxv = j * 16 + iota
                a0 = plsc.load_gather(cch[0], [idxv])
                a1 = plsc.load_gather(cch[1], [idxv])
                a2 = plsc.load_gather(cch[2], [idxv])
                a3 = plsc.load_gather(cch[3], [idxv])
                sc = plsc.load_gather(cch[5], [idxv])
                yy1 = jnp.maximum(wy1v, a0)
                xx1 = jnp.maximum(wx1v, a1)
                yy2 = jnp.minimum(wy2v, a2)
                xx2 = jnp.minimum(wx2v, a3)
                inter = jnp.maximum(yy2 - yy1, 0.0) * jnp.maximum(
                    xx2 - xx1, 0.0
                )
                area = (a2 - a0) * (a3 - a1)
                iou = inter / (wareav + area - inter + 1e-8)
                newsc = jnp.where(iou > _NMS_T, -1.0, sc)
                plsc.store_scatter(cch[5], [idxv], newsc)
                return 0

            lax.fori_loop(0, nchunks, sup_body, 0)

            own = (win_slot == slot) & (ii < _BIG)
            plsc.store_scatter(cch[5], [iis], neg16, mask=(iota == 0) & own)

            @pl.when(slot == 0)
            def _():
                hot = iota == 0
                di = jnp.full((16,), i, jnp.int32)
                offv = wcls * 4.0
                vals = (
                    wy1 - offv,
                    wx1 - offv,
                    wy2 - offv,
                    wx2 - offv,
                    wcls,
                    win_sc,
                )
                for k in range(6):
                    plsc.store_scatter(
                        det[k], [di], jnp.full((16,), vals[k]), mask=hot
                    )

        plsc.subcore_barrier()
        return 0

    lax.fori_loop(0, _MAXDET, nms_body, 0)

    @pl.when(slot == 0)
    def _():
        for k in range(6):
            pltpu.sync_copy(det[k], out_hbm.at[batch, k])


def kernel(rois, classifications):
    rois_t = jnp.transpose(rois, (0, 2, 1))
    cls_t = jnp.transpose(classifications, (0, 2, 1))
    pad = _NPAD - _N
    rois_t = jnp.pad(rois_t, ((0, 0), (0, 0), (0, pad)))
    cls_t = jnp.pad(cls_t, ((0, 0), (0, 0), (0, pad)))
    rois_t = rois_t.reshape(_B, 4, _BLKS, 128)
    cls_t = cls_t.reshape(_B, 6, _BLKS, 128)

    cand = pl.pallas_call(
        _prep_kernel,
        out_shape=jax.ShapeDtypeStruct((_B, 6, _BLKS, 128), jnp.float32),
    )(rois_t, cls_t)
    cand = cand.reshape(_B, 6, _NPAD)

    mesh = plsc.VectorSubcoreMesh(core_axis_name="c", subcore_axis_name="s")
    scratch = (
        [pltpu.VMEM((_SEG,), jnp.float32) for _ in range(6)]
        + [pltpu.VMEM((_CAP,), jnp.float32) for _ in range(6)]
        + [pltpu.VMEM((16,), jnp.float32)]
        + [pltpu.VMEM((128,), jnp.float32)]
        + [pltpu.VMEM((128,), jnp.float32) for _ in range(6)]
        + [pltpu.VMEM_SHARED((256,), jnp.float32)]
    )
    nms = functools.partial(
        pl.kernel,
        mesh=mesh,
        out_type=jax.ShapeDtypeStruct((_B, 6, 128), jnp.float32),
        scratch_types=scratch,
        compiler_params=pltpu.CompilerParams(needs_layout_passes=False),
    )(_nms_sc)
    out = nms(cand)
    return jnp.transpose(out[:, :, :_MAXDET], (0, 2, 1))
